# Initial kernel scaffold; baseline (speedup 1.0000x reference)
import jax
import jax.numpy as jnp
from jax.experimental import pallas as pl


def kernel(x, edge_index, edge_attr, batch, W_msg_w, W_msg_b, W_apply_w, W_apply_b, ln_g, ln_b, pred_w, pred_b):
    raise NotImplementedError("write your pallas kernel here")


# speedup vs baseline: 2.5586x; 2.5586x over previous
import functools

import jax
import jax.numpy as jnp
from jax import lax
from jax.experimental import pallas as pl
from jax.experimental.pallas import tpu as pltpu
from jax.experimental.pallas import tpu_sc as plsc

N = 10000
E = 320000
D = 128
DE = 16
H1 = 128
PH = 128

NC = 2
NS = 16
NW = NC * NS
L = 16
NH = 4
DQ = D // NH

NPAD = 10240
ROWS_PER_TILE = NPAD // NS
B = 128
EPT = -(-E // NW)
NB = -(-EPT // B)
EPT_PAD = NB * B
EPAD = NW * EPT_PAD


def _sc_mesh():
    return plsc.VectorSubcoreMesh(
        core_axis_name="c", subcore_axis_name="s",
        num_cores=NC, num_subcores=NS)


def _newton_rsqrt(d):
    yi = 0x5F3759DF - lax.shift_right_logical(
        lax.bitcast_convert_type(d, jnp.int32), 1)
    y = lax.bitcast_convert_type(yi, jnp.float32)
    for _ in range(3):
        y = y * (1.5 - 0.5 * d * y * y)
    return y


DW = 2 * DE
CHUNK = 128


@functools.cache
def _make_sc_degree_attr():
    @functools.partial(
        pl.kernel,
        out_type=jax.ShapeDtypeStruct((NC, NPAD, DW), jnp.float32),
        mesh=_sc_mesh(),
        compiler_params=pltpu.CompilerParams(needs_layout_passes=False),
        scratch_types=[
            pltpu.VMEM((NB, B), jnp.int32),
            pltpu.VMEM((16, 128), jnp.float32),
            pltpu.VMEM((B, DW), jnp.float32),
            pltpu.VMEM((B,), jnp.int32),
            pltpu.VMEM_SHARED((NPAD, DW), jnp.float32),
            pltpu.SemaphoreType.DMA,
            pltpu.SemaphoreType.DMA,
        ],
    )
    def sc_degree_attr(rc_hbm, eapk_hbm, attr_out,
                       idx_v, epk_v, ea_v, colb_v,
                       attr_sh, sem_a, sem_b):
        c = lax.axis_index("c")
        s = lax.axis_index("s")
        w = s * NC + c
        my_rows = s * ROWS_PER_TILE

        z = jnp.zeros((L,), jnp.float32)

        @pl.loop(0, B)
        def _(r):
            ea_v[r, pl.ds(0, L)] = z
            ea_v[r, pl.ds(L, L)] = z

        for t in range(ROWS_PER_TILE // B):
            pltpu.sync_copy(ea_v, attr_sh.at[pl.ds(my_rows + t * B, B)])
        plsc.subcore_barrier()

        one_hot = jnp.where(lax.iota(jnp.int32, L) == 0, 1.0, 0.0)

        @pl.loop(0, B)
        def _(r):
            ea_v[r, pl.ds(L, L)] = one_hot

        @pl.loop(0, NB)
        def _(i):
            cp1 = pltpu.async_copy(
                eapk_hbm.at[pl.ds(w * (EPT_PAD // 8) + i * (B // 8), B // 8)],
                epk_v, sem_a)
            cp2 = pltpu.async_copy(rc_hbm.at[w, 1, i], colb_v, sem_b)
            cp1.wait()
            cp2.wait()
            for r in range(B):
                ea_v[r, pl.ds(0, L)] = epk_v[r // 8, pl.ds((r % 8) * L, L)]
            pltpu.async_copy(ea_v, attr_sh.at[colb_v], sem_a,
                             add=True).wait()

        @pl.loop(0, B)
        def _(r):
            ea_v[r, pl.ds(0, L)] = z

        ow = s * NC + (1 - c)

        @pl.loop(0, NB)
        def _(i):
            pltpu.async_copy(rc_hbm.at[ow, 1, i], colb_v, sem_b).wait()
            pltpu.async_copy(ea_v, attr_sh.at[colb_v], sem_a,
                             add=True).wait()

        plsc.subcore_barrier()
        pltpu.sync_copy(attr_sh.at[pl.ds(my_rows, ROWS_PER_TILE)],
                        attr_out.at[c, pl.ds(my_rows, ROWS_PER_TILE)])

    return sc_degree_attr


@functools.cache
def _make_sc_normrow():
    @functools.partial(
        pl.kernel,
        out_type=jax.ShapeDtypeStruct((NW, NB, B), jnp.float32),
        mesh=_sc_mesh(),
        compiler_params=pltpu.CompilerParams(needs_layout_passes=False),
        scratch_types=[
            pltpu.VMEM((NB, B), jnp.int32),
            pltpu.VMEM((NPAD,), jnp.float32),
            pltpu.VMEM((B,), jnp.float32),
            pltpu.SemaphoreType.DMA,
        ],
    )
    def sc_normrow(rc_hbm, norm_hbm, nr_out, idx_v, norm_f, nbuf_v, sem_a):
        c = lax.axis_index("c")
        s = lax.axis_index("s")
        w = s * NC + c
        pltpu.sync_copy(norm_hbm, norm_f)
        pltpu.sync_copy(rc_hbm.at[w, 0], idx_v)

        @pl.loop(0, NB)
        def _(i):
            for j in range(B // L):
                ridx = idx_v[i, pl.ds(j * L, L)]
                nbuf_v[pl.ds(j * L, L)] = plsc.load_gather(norm_f, [ridx])
            pltpu.async_copy(nbuf_v, nr_out.at[w, i], sem_a).wait()

    return sc_normrow


@functools.cache
def _make_sc_message_scatter():
    @functools.partial(
        pl.kernel,
        out_type=jax.ShapeDtypeStruct((NC, NH, NPAD, DQ), jnp.float32),
        mesh=_sc_mesh(),
        compiler_params=pltpu.CompilerParams(needs_layout_passes=False),
        scratch_types=[
            pltpu.VMEM((B,), jnp.int32),
            pltpu.VMEM((B,), jnp.int32),
            pltpu.VMEM((B, D), jnp.float32),
            pltpu.VMEM((B // 4, 128), jnp.float32),
            pltpu.VMEM((B, DQ), jnp.float32),
            pltpu.VMEM_SHARED((NPAD, DQ), jnp.float32),
            pltpu.SemaphoreType.DMA,
            pltpu.SemaphoreType.DMA,
        ],
    )
    def sc_message_scatter(xwb_hbm, ewpk_hbm, rc_hbm, s_out,
                           ridx_v, cidx_v, gbuf, ebuf, mbuf,
                           s_sh, sem_a, sem_b):
        c = lax.axis_index("c")
        s = lax.axis_index("s")
        w = s * NC + c
        my_rows = s * ROWS_PER_TILE

        lane = lax.iota(jnp.int32, L)
        z = jnp.zeros((L,), jnp.float32)

        for h in range(NH):
            @pl.loop(0, B)
            def _(r):
                for f in range(DQ // L):
                    mbuf[r, pl.ds(f * L, L)] = z

            for t in range(ROWS_PER_TILE // B):
                pltpu.sync_copy(mbuf, s_sh.at[pl.ds(my_rows + t * B, B)])
            plsc.subcore_barrier()

            @pl.loop(0, NB)
            def _(i):
                cpr = pltpu.async_copy(rc_hbm.at[w, 0, i], ridx_v, sem_a)
                cpc = pltpu.async_copy(rc_hbm.at[w, 1, i], cidx_v, sem_b)
                cpr.wait()
                gather = pltpu.async_copy(xwb_hbm.at[ridx_v], gbuf, sem_a)
                base = (h * (EPAD // 4)
                        + w * (EPT_PAD // 4) + i * (B // 4))
                pltpu.sync_copy(ewpk_hbm.at[pl.ds(base, B // 4)], ebuf)
                cpc.wait()
                gather.wait()

                for e in range(B):
                    for f in range(DQ // L):
                        zv = (gbuf[e, pl.ds(h * DQ + f * L, L)]
                              + ebuf[e // 4,
                                     pl.ds((e % 4) * DQ + f * L, L)])
                        mbuf[e, pl.ds(f * L, L)] = jnp.maximum(zv, 0.0)

                pltpu.sync_copy(mbuf, s_sh.at[cidx_v], add=True)

            plsc.subcore_barrier()
            pltpu.sync_copy(
                s_sh.at[pl.ds(my_rows, ROWS_PER_TILE)],
                s_out.at[c, h, pl.ds(my_rows, ROWS_PER_TILE)])
            plsc.subcore_barrier()

    return sc_message_scatter


def _tc_xwb_body(x_ref, wx_ref, b_ref, out_ref):
    out_ref[...] = lax.dot_general(
        x_ref[...], wx_ref[...], (((1,), (1,)), ((), ())),
        preferred_element_type=jnp.float32) + b_ref[...]


def _tc_eaw_body(ea_ref, we_ref, out_ref):
    out_ref[...] = lax.dot_general(
        ea_ref[...], we_ref[...], (((1,), (1,)), ((), ())),
        preferred_element_type=jnp.float32)


def _tc_apply_body(x_ref, xwb_ref, la_ref, nrm_ref, s_ref,
                   we_ref, wa_ref, ba_ref, out_ref):
    p = pl.program_id(0)
    selfz = xwb_ref[...] + lax.dot_general(
        la_ref[...], we_ref[...], (((1,), (1,)), ((), ())),
        preferred_element_type=jnp.float32)
    nrm = nrm_ref[...]
    aggr = nrm * s_ref[...] + nrm * nrm * jnp.maximum(selfz, 0.0)
    xa = jnp.concatenate([x_ref[...], aggr], axis=1)
    h = jnp.maximum(
        lax.dot_general(xa, wa_ref[...], (((1,), (1,)), ((), ())),
                        preferred_element_type=jnp.float32) + ba_ref[...],
        0.0)
    rowid = p * 128 + lax.broadcasted_iota(jnp.int32, (128, 1), 0)
    h = jnp.where(rowid < N, h, 0.0)

    @pl.when(p == 0)
    def _():
        out_ref[...] = jnp.zeros_like(out_ref)

    out_ref[...] += jnp.sum(h, axis=0, keepdims=True)


def _tc_head_body(sum_ref, lng_ref, lnb_ref, pw_ref, pb_ref, out_ref):
    gf = sum_ref[...] / jnp.float32(N)
    mu = jnp.mean(gf, axis=-1, keepdims=True)
    var = jnp.mean((gf - mu) ** 2, axis=-1, keepdims=True)
    g2 = (gf - mu) * lax.rsqrt(var + 1e-5) * lng_ref[...] + lnb_ref[...]
    out_ref[...] = jnp.maximum(
        lax.dot_general(g2, pw_ref[...], (((1,), (1,)), ((), ())),
                        preferred_element_type=jnp.float32) + pb_ref[...],
        0.0)


def kernel(x, edge_index, edge_attr, batch, W_msg_w, W_msg_b,
           W_apply_w, W_apply_b, ln_g, ln_b, pred_w, pred_b):
    f32 = jnp.float32
    row0 = edge_index[0]
    col0 = edge_index[1]
    pad_e = EPAD - E

    row_p = jnp.concatenate([row0, jnp.zeros((pad_e,), jnp.int32)])
    col_p = jnp.concatenate([col0, jnp.full((pad_e,), N, jnp.int32)])
    rc1 = jnp.stack([row_p.reshape(NW, EPT_PAD), col_p.reshape(NW, EPT_PAD)],
                    axis=1).reshape(NW, 2, NB, B)
    ea_p = jnp.concatenate([edge_attr, jnp.zeros((pad_e, DE), f32)], axis=0)
    ea_pk = ea_p.reshape(EPAD // 8, 128)
    x_p = jnp.concatenate([x, jnp.zeros((NPAD - N, D), f32)], axis=0)
    Wx = W_msg_w[:, :D]
    We = W_msg_w[:, D:]
    b2 = W_msg_b.reshape(1, D)

    attr_part = _make_sc_degree_attr()(rc1, ea_pk)
    cnt = attr_part[0, :, DE]
    attr_sum = attr_part[0, :, :DE] + attr_part[1, :, :DE]
    deg = cnt + 1.0
    norm = lax.rsqrt(deg)
    loop_attr = attr_sum / jnp.maximum(cnt, 1.0)[:, None]
    nr = _make_sc_normrow()(rc1, norm)

    xwb = pl.pallas_call(
        _tc_xwb_body,
        grid=(NPAD // 256,),
        in_specs=[
            pl.BlockSpec((256, D), lambda p: (p, 0)),
            pl.BlockSpec((D, D), lambda p: (0, 0)),
            pl.BlockSpec((1, D), lambda p: (0, 0)),
        ],
        out_specs=pl.BlockSpec((256, D), lambda p: (p, 0)),
        out_shape=jax.ShapeDtypeStruct((NPAD, D), f32),
    )(x_p, Wx, b2)

    eaw = pl.pallas_call(
        _tc_eaw_body,
        grid=(EPAD // 1024,),
        in_specs=[
            pl.BlockSpec((1024, DE), lambda p: (p, 0)),
            pl.BlockSpec((D, DE), lambda p: (0, 0)),
        ],
        out_specs=pl.BlockSpec((1024, D), lambda p: (p, 0)),
        out_shape=jax.ShapeDtypeStruct((EPAD, D), f32),
    )(ea_p, We)
    xwn = xwb * norm[:, None]
    eawn = eaw * nr.reshape(EPAD)[:, None]
    ew_pk = jnp.transpose(
        eawn.reshape(EPAD, NH, DQ), (1, 0, 2)).reshape(NH * (EPAD // 4), 128)

    s_part = _make_sc_message_scatter()(xwn, ew_pk, rc1)
    s_sum = s_part[0] + s_part[1]
    s_full = jnp.transpose(s_sum, (1, 0, 2)).reshape(NPAD, D)

    hsum = pl.pallas_call(
        _tc_apply_body,
        grid=(NPAD // 128,),
        in_specs=[
            pl.BlockSpec((128, D), lambda p: (p, 0)),
            pl.BlockSpec((128, D), lambda p: (p, 0)),
            pl.BlockSpec((128, DE), lambda p: (p, 0)),
            pl.BlockSpec((128, 1), lambda p: (p, 0)),
            pl.BlockSpec((128, D), lambda p: (p, 0)),
            pl.BlockSpec((D, DE), lambda p: (0, 0)),
            pl.BlockSpec((H1, 2 * D), lambda p: (0, 0)),
            pl.BlockSpec((1, H1), lambda p: (0, 0)),
        ],
        out_specs=pl.BlockSpec((1, H1), lambda p: (0, 0)),
        out_shape=jax.ShapeDtypeStruct((1, H1), f32),
    )(x_p, xwb, loop_attr, norm[:, None], s_full,
      We, W_apply_w, W_apply_b.reshape(1, H1))

    out = pl.pallas_call(
        _tc_head_body,
        in_specs=[
            pl.BlockSpec((1, H1), lambda: (0, 0)),
            pl.BlockSpec((1, H1), lambda: (0, 0)),
            pl.BlockSpec((1, H1), lambda: (0, 0)),
            pl.BlockSpec((PH, H1), lambda: (0, 0)),
            pl.BlockSpec((1, PH), lambda: (0, 0)),
        ],
        out_specs=pl.BlockSpec((1, PH), lambda: (0, 0)),
        out_shape=jax.ShapeDtypeStruct((1, PH), f32),
    )(hsum, ln_g.reshape(1, H1), ln_b.reshape(1, H1),
      pred_w, pred_b.reshape(1, PH))

    return out


# speedup vs baseline: 2.6650x; 1.0416x over previous
import functools

import jax
import jax.numpy as jnp
from jax import lax
from jax.experimental import pallas as pl
from jax.experimental.pallas import tpu as pltpu
from jax.experimental.pallas import tpu_sc as plsc

N = 10000
E = 320000
D = 128
DE = 16
H1 = 128
PH = 128

NC = 2
NS = 16
NW = NC * NS
L = 16
NH = 4
DQ = D // NH

NPAD = 10240
ROWS_PER_TILE = NPAD // NS
B = 128
EPT = -(-E // NW)
NB = -(-EPT // B)
EPT_PAD = NB * B
EPAD = NW * EPT_PAD


def _sc_mesh():
    return plsc.VectorSubcoreMesh(
        core_axis_name="c", subcore_axis_name="s",
        num_cores=NC, num_subcores=NS)


def _newton_rsqrt(d):
    yi = 0x5F3759DF - lax.shift_right_logical(
        lax.bitcast_convert_type(d, jnp.int32), 1)
    y = lax.bitcast_convert_type(yi, jnp.float32)
    for _ in range(3):
        y = y * (1.5 - 0.5 * d * y * y)
    return y


DW = 2 * DE
CHUNK = 128


@functools.cache
def _make_sc_degree_attr():
    @functools.partial(
        pl.kernel,
        out_type=jax.ShapeDtypeStruct((NC, NPAD, DW), jnp.float32),
        mesh=_sc_mesh(),
        compiler_params=pltpu.CompilerParams(needs_layout_passes=False),
        scratch_types=[
            pltpu.VMEM((NB, B), jnp.int32),
            pltpu.VMEM((16, 128), jnp.float32),
            pltpu.VMEM((B, DW), jnp.float32),
            pltpu.VMEM((B,), jnp.int32),
            pltpu.VMEM_SHARED((NPAD, DW), jnp.float32),
            pltpu.SemaphoreType.DMA,
            pltpu.SemaphoreType.DMA,
        ],
    )
    def sc_degree_attr(rc_hbm, eapk_hbm, attr_out,
                       idx_v, epk_v, ea_v, colb_v,
                       attr_sh, sem_a, sem_b):
        c = lax.axis_index("c")
        s = lax.axis_index("s")
        w = s * NC + c
        my_rows = s * ROWS_PER_TILE

        z = jnp.zeros((L,), jnp.float32)

        @pl.loop(0, B)
        def _(r):
            ea_v[r, pl.ds(0, L)] = z
            ea_v[r, pl.ds(L, L)] = z

        for t in range(ROWS_PER_TILE // B):
            pltpu.sync_copy(ea_v, attr_sh.at[pl.ds(my_rows + t * B, B)])
        plsc.subcore_barrier()

        one_hot = jnp.where(lax.iota(jnp.int32, L) == 0, 1.0, 0.0)

        @pl.loop(0, B)
        def _(r):
            ea_v[r, pl.ds(L, L)] = one_hot

        @pl.loop(0, NB)
        def _(i):
            cp1 = pltpu.async_copy(
                eapk_hbm.at[pl.ds(w * (EPT_PAD // 8) + i * (B // 8), B // 8)],
                epk_v, sem_a)
            cp2 = pltpu.async_copy(rc_hbm.at[w, 1, i], colb_v, sem_b)
            cp1.wait()
            cp2.wait()
            for r in range(B):
                ea_v[r, pl.ds(0, L)] = epk_v[r // 8, pl.ds((r % 8) * L, L)]
            pltpu.async_copy(ea_v, attr_sh.at[colb_v], sem_a,
                             add=True).wait()

        @pl.loop(0, B)
        def _(r):
            ea_v[r, pl.ds(0, L)] = z

        ow = s * NC + (1 - c)

        @pl.loop(0, NB)
        def _(i):
            pltpu.async_copy(rc_hbm.at[ow, 1, i], colb_v, sem_b).wait()
            pltpu.async_copy(ea_v, attr_sh.at[colb_v], sem_a,
                             add=True).wait()

        plsc.subcore_barrier()
        pltpu.sync_copy(attr_sh.at[pl.ds(my_rows, ROWS_PER_TILE)],
                        attr_out.at[c, pl.ds(my_rows, ROWS_PER_TILE)])

    return sc_degree_attr


@functools.cache
def _make_sc_normrow():
    @functools.partial(
        pl.kernel,
        out_type=jax.ShapeDtypeStruct((NW, NB, B), jnp.float32),
        mesh=_sc_mesh(),
        compiler_params=pltpu.CompilerParams(needs_layout_passes=False),
        scratch_types=[
            pltpu.VMEM((NB, B), jnp.int32),
            pltpu.VMEM((NPAD,), jnp.float32),
            pltpu.VMEM((B,), jnp.float32),
            pltpu.SemaphoreType.DMA,
        ],
    )
    def sc_normrow(rc_hbm, norm_hbm, nr_out, idx_v, norm_f, nbuf_v, sem_a):
        c = lax.axis_index("c")
        s = lax.axis_index("s")
        w = s * NC + c
        pltpu.sync_copy(norm_hbm, norm_f)
        pltpu.sync_copy(rc_hbm.at[w, 0], idx_v)

        @pl.loop(0, NB)
        def _(i):
            for j in range(B // L):
                ridx = idx_v[i, pl.ds(j * L, L)]
                nbuf_v[pl.ds(j * L, L)] = plsc.load_gather(norm_f, [ridx])
            pltpu.async_copy(nbuf_v, nr_out.at[w, i], sem_a).wait()

    return sc_normrow


@functools.cache
def _make_sc_message_scatter():
    @functools.partial(
        pl.kernel,
        out_type=jax.ShapeDtypeStruct((NC, NH, NPAD, DQ), jnp.float32),
        mesh=_sc_mesh(),
        compiler_params=pltpu.CompilerParams(needs_layout_passes=False),
        scratch_types=[
            pltpu.VMEM((NB, B), jnp.int32),
            pltpu.VMEM((B,), jnp.int32),
            pltpu.VMEM((B, D), jnp.float32),
            pltpu.VMEM((B // 4, 128), jnp.float32),
            pltpu.VMEM((B, DQ), jnp.float32),
            pltpu.VMEM_SHARED((NPAD, DQ), jnp.float32),
            pltpu.SemaphoreType.DMA,
            pltpu.SemaphoreType.DMA,
        ],
    )
    def sc_message_scatter(xwb_hbm, ewpk_hbm, rc_hbm, s_out,
                           ridx_v, cidx_v, gbuf, ebuf, mbuf,
                           s_sh, sem_a, sem_b):
        c = lax.axis_index("c")
        s = lax.axis_index("s")
        w = s * NC + c
        my_rows = s * ROWS_PER_TILE

        pltpu.sync_copy(rc_hbm.at[w, 0], ridx_v)
        z = jnp.zeros((L,), jnp.float32)

        for h in range(NH):
            @pl.loop(0, B)
            def _(r):
                for f in range(DQ // L):
                    mbuf[r, pl.ds(f * L, L)] = z

            for t in range(ROWS_PER_TILE // B):
                pltpu.sync_copy(mbuf, s_sh.at[pl.ds(my_rows + t * B, B)])
            plsc.subcore_barrier()

            @pl.loop(0, NB)
            def _(i):
                cpc = pltpu.async_copy(rc_hbm.at[w, 1, i], cidx_v, sem_b)
                gather = pltpu.async_copy(
                    xwb_hbm.at[ridx_v.at[i]], gbuf, sem_a)
                base = (h * (EPAD // 4)
                        + w * (EPT_PAD // 4) + i * (B // 4))
                pltpu.sync_copy(ewpk_hbm.at[pl.ds(base, B // 4)], ebuf)
                cpc.wait()
                gather.wait()

                for e in range(B):
                    for f in range(DQ // L):
                        zv = (gbuf[e, pl.ds(h * DQ + f * L, L)]
                              + ebuf[e // 4,
                                     pl.ds((e % 4) * DQ + f * L, L)])
                        mbuf[e, pl.ds(f * L, L)] = jnp.maximum(zv, 0.0)

                pltpu.sync_copy(mbuf, s_sh.at[cidx_v], add=True)

            plsc.subcore_barrier()
            pltpu.sync_copy(
                s_sh.at[pl.ds(my_rows, ROWS_PER_TILE)],
                s_out.at[c, h, pl.ds(my_rows, ROWS_PER_TILE)])
            plsc.subcore_barrier()

    return sc_message_scatter


def _tc_xwb_body(x_ref, wx_ref, b_ref, out_ref):
    out_ref[...] = lax.dot_general(
        x_ref[...], wx_ref[...], (((1,), (1,)), ((), ())),
        preferred_element_type=jnp.float32) + b_ref[...]


def _tc_eaw_body(ea_ref, we_ref, out_ref):
    out_ref[...] = lax.dot_general(
        ea_ref[...], we_ref[...], (((1,), (1,)), ((), ())),
        preferred_element_type=jnp.float32)


def _tc_apply_body(x_ref, xwb_ref, la_ref, nrm_ref, s_ref,
                   we_ref, wa_ref, ba_ref, out_ref):
    p = pl.program_id(0)
    selfz = xwb_ref[...] + lax.dot_general(
        la_ref[...], we_ref[...], (((1,), (1,)), ((), ())),
        preferred_element_type=jnp.float32)
    nrm = nrm_ref[...]
    aggr = nrm * s_ref[...] + nrm * nrm * jnp.maximum(selfz, 0.0)
    xa = jnp.concatenate([x_ref[...], aggr], axis=1)
    h = jnp.maximum(
        lax.dot_general(xa, wa_ref[...], (((1,), (1,)), ((), ())),
                        preferred_element_type=jnp.float32) + ba_ref[...],
        0.0)
    rowid = p * 128 + lax.broadcasted_iota(jnp.int32, (128, 1), 0)
    h = jnp.where(rowid < N, h, 0.0)

    @pl.when(p == 0)
    def _():
        out_ref[...] = jnp.zeros_like(out_ref)

    out_ref[...] += jnp.sum(h, axis=0, keepdims=True)


def _tc_head_body(sum_ref, lng_ref, lnb_ref, pw_ref, pb_ref, out_ref):
    gf = sum_ref[...] / jnp.float32(N)
    mu = jnp.mean(gf, axis=-1, keepdims=True)
    var = jnp.mean((gf - mu) ** 2, axis=-1, keepdims=True)
    g2 = (gf - mu) * lax.rsqrt(var + 1e-5) * lng_ref[...] + lnb_ref[...]
    out_ref[...] = jnp.maximum(
        lax.dot_general(g2, pw_ref[...], (((1,), (1,)), ((), ())),
                        preferred_element_type=jnp.float32) + pb_ref[...],
        0.0)


def kernel(x, edge_index, edge_attr, batch, W_msg_w, W_msg_b,
           W_apply_w, W_apply_b, ln_g, ln_b, pred_w, pred_b):
    f32 = jnp.float32
    row0 = edge_index[0]
    col0 = edge_index[1]
    pad_e = EPAD - E

    row_p = jnp.concatenate([row0, jnp.zeros((pad_e,), jnp.int32)])
    col_p = jnp.concatenate([col0, jnp.full((pad_e,), N, jnp.int32)])
    rc1 = jnp.stack([row_p.reshape(NW, EPT_PAD), col_p.reshape(NW, EPT_PAD)],
                    axis=1).reshape(NW, 2, NB, B)
    ea_p = jnp.concatenate([edge_attr, jnp.zeros((pad_e, DE), f32)], axis=0)
    ea_pk = ea_p.reshape(EPAD // 8, 128)
    x_p = jnp.concatenate([x, jnp.zeros((NPAD - N, D), f32)], axis=0)
    Wx = W_msg_w[:, :D]
    We = W_msg_w[:, D:]
    b2 = W_msg_b.reshape(1, D)

    attr_part = _make_sc_degree_attr()(rc1, ea_pk)
    cnt = attr_part[0, :, DE]
    attr_sum = attr_part[0, :, :DE] + attr_part[1, :, :DE]
    deg = cnt + 1.0
    norm = lax.rsqrt(deg)
    loop_attr = attr_sum / jnp.maximum(cnt, 1.0)[:, None]
    nr = _make_sc_normrow()(rc1, norm)

    xwb = pl.pallas_call(
        _tc_xwb_body,
        grid=(NPAD // 256,),
        in_specs=[
            pl.BlockSpec((256, D), lambda p: (p, 0)),
            pl.BlockSpec((D, D), lambda p: (0, 0)),
            pl.BlockSpec((1, D), lambda p: (0, 0)),
        ],
        out_specs=pl.BlockSpec((256, D), lambda p: (p, 0)),
        out_shape=jax.ShapeDtypeStruct((NPAD, D), f32),
    )(x_p, Wx, b2)

    eaw = pl.pallas_call(
        _tc_eaw_body,
        grid=(EPAD // 1024,),
        in_specs=[
            pl.BlockSpec((1024, DE), lambda p: (p, 0)),
            pl.BlockSpec((D, DE), lambda p: (0, 0)),
        ],
        out_specs=pl.BlockSpec((1024, D), lambda p: (p, 0)),
        out_shape=jax.ShapeDtypeStruct((EPAD, D), f32),
    )(ea_p, We)
    xwn = xwb * norm[:, None]
    eawn = eaw * nr.reshape(EPAD)[:, None]
    ew_pk = jnp.transpose(
        eawn.reshape(EPAD, NH, DQ), (1, 0, 2)).reshape(NH * (EPAD // 4), 128)

    s_part = _make_sc_message_scatter()(xwn, ew_pk, rc1)
    s_sum = s_part[0] + s_part[1]
    s_full = jnp.transpose(s_sum, (1, 0, 2)).reshape(NPAD, D)

    hsum = pl.pallas_call(
        _tc_apply_body,
        grid=(NPAD // 128,),
        in_specs=[
            pl.BlockSpec((128, D), lambda p: (p, 0)),
            pl.BlockSpec((128, D), lambda p: (p, 0)),
            pl.BlockSpec((128, DE), lambda p: (p, 0)),
            pl.BlockSpec((128, 1), lambda p: (p, 0)),
            pl.BlockSpec((128, D), lambda p: (p, 0)),
            pl.BlockSpec((D, DE), lambda p: (0, 0)),
            pl.BlockSpec((H1, 2 * D), lambda p: (0, 0)),
            pl.BlockSpec((1, H1), lambda p: (0, 0)),
        ],
        out_specs=pl.BlockSpec((1, H1), lambda p: (0, 0)),
        out_shape=jax.ShapeDtypeStruct((1, H1), f32),
    )(x_p, xwb, loop_attr, norm[:, None], s_full,
      We, W_apply_w, W_apply_b.reshape(1, H1))

    out = pl.pallas_call(
        _tc_head_body,
        in_specs=[
            pl.BlockSpec((1, H1), lambda: (0, 0)),
            pl.BlockSpec((1, H1), lambda: (0, 0)),
            pl.BlockSpec((1, H1), lambda: (0, 0)),
            pl.BlockSpec((PH, H1), lambda: (0, 0)),
            pl.BlockSpec((1, PH), lambda: (0, 0)),
        ],
        out_specs=pl.BlockSpec((1, PH), lambda: (0, 0)),
        out_shape=jax.ShapeDtypeStruct((1, PH), f32),
    )(hsum, ln_g.reshape(1, H1), ln_b.reshape(1, H1),
      pred_w, pred_b.reshape(1, PH))

    return out


# speedup vs baseline: 2.6692x; 1.0015x over previous
import functools

import jax
import jax.numpy as jnp
from jax import lax
from jax.experimental import pallas as pl
from jax.experimental.pallas import tpu as pltpu
from jax.experimental.pallas import tpu_sc as plsc

N = 10000
E = 320000
D = 128
DE = 16
H1 = 128
PH = 128

NC = 2
NS = 16
NW = NC * NS
L = 16
NH = 4
DQ = D // NH

NPAD = 10240
ROWS_PER_TILE = NPAD // NS
B = 128
EPT = -(-E // NW)
NB = -(-EPT // B)
EPT_PAD = NB * B
EPAD = NW * EPT_PAD


def _sc_mesh():
    return plsc.VectorSubcoreMesh(
        core_axis_name="c", subcore_axis_name="s",
        num_cores=NC, num_subcores=NS)


def _newton_rsqrt(d):
    yi = 0x5F3759DF - lax.shift_right_logical(
        lax.bitcast_convert_type(d, jnp.int32), 1)
    y = lax.bitcast_convert_type(yi, jnp.float32)
    for _ in range(3):
        y = y * (1.5 - 0.5 * d * y * y)
    return y


DW = 2 * DE
CHUNK = 128


@functools.cache
def _make_sc_degree_attr():
    @functools.partial(
        pl.kernel,
        out_type=jax.ShapeDtypeStruct((NC, NPAD, DW), jnp.float32),
        mesh=_sc_mesh(),
        compiler_params=pltpu.CompilerParams(needs_layout_passes=False),
        scratch_types=[
            pltpu.VMEM((NB, B), jnp.int32),
            pltpu.VMEM((16, 128), jnp.float32),
            pltpu.VMEM((B, DW), jnp.float32),
            pltpu.VMEM((B,), jnp.int32),
            pltpu.VMEM_SHARED((NPAD, DW), jnp.float32),
            pltpu.SemaphoreType.DMA,
            pltpu.SemaphoreType.DMA,
        ],
    )
    def sc_degree_attr(rc_hbm, eapk_hbm, attr_out,
                       idx_v, epk_v, ea_v, colb_v,
                       attr_sh, sem_a, sem_b):
        c = lax.axis_index("c")
        s = lax.axis_index("s")
        w = s * NC + c
        my_rows = s * ROWS_PER_TILE

        z = jnp.zeros((L,), jnp.float32)

        @pl.loop(0, B)
        def _(r):
            ea_v[r, pl.ds(0, L)] = z
            ea_v[r, pl.ds(L, L)] = z

        for t in range(ROWS_PER_TILE // B):
            pltpu.sync_copy(ea_v, attr_sh.at[pl.ds(my_rows + t * B, B)])
        plsc.subcore_barrier()

        one_hot = jnp.where(lax.iota(jnp.int32, L) == 0, 1.0, 0.0)

        @pl.loop(0, B)
        def _(r):
            ea_v[r, pl.ds(L, L)] = one_hot

        @pl.loop(0, NB)
        def _(i):
            cp1 = pltpu.async_copy(
                eapk_hbm.at[pl.ds(w * (EPT_PAD // 8) + i * (B // 8), B // 8)],
                epk_v, sem_a)
            cp2 = pltpu.async_copy(rc_hbm.at[w, 1, i], colb_v, sem_b)
            cp1.wait()
            cp2.wait()
            for r in range(B):
                ea_v[r, pl.ds(0, L)] = epk_v[r // 8, pl.ds((r % 8) * L, L)]
            pltpu.async_copy(ea_v, attr_sh.at[colb_v], sem_a,
                             add=True).wait()

        @pl.loop(0, B)
        def _(r):
            ea_v[r, pl.ds(0, L)] = z

        ow = s * NC + (1 - c)

        @pl.loop(0, NB)
        def _(i):
            pltpu.async_copy(rc_hbm.at[ow, 1, i], colb_v, sem_b).wait()
            pltpu.async_copy(ea_v, attr_sh.at[colb_v], sem_a,
                             add=True).wait()

        plsc.subcore_barrier()
        pltpu.sync_copy(attr_sh.at[pl.ds(my_rows, ROWS_PER_TILE)],
                        attr_out.at[c, pl.ds(my_rows, ROWS_PER_TILE)])

    return sc_degree_attr


@functools.cache
def _make_sc_normrow():
    @functools.partial(
        pl.kernel,
        out_type=jax.ShapeDtypeStruct((NW, NB, B), jnp.float32),
        mesh=_sc_mesh(),
        compiler_params=pltpu.CompilerParams(needs_layout_passes=False),
        scratch_types=[
            pltpu.VMEM((NB, B), jnp.int32),
            pltpu.VMEM((NPAD,), jnp.float32),
            pltpu.VMEM((B,), jnp.float32),
            pltpu.SemaphoreType.DMA,
        ],
    )
    def sc_normrow(rc_hbm, norm_hbm, nr_out, idx_v, norm_f, nbuf_v, sem_a):
        c = lax.axis_index("c")
        s = lax.axis_index("s")
        w = s * NC + c
        pltpu.sync_copy(norm_hbm, norm_f)
        pltpu.sync_copy(rc_hbm.at[w, 0], idx_v)

        @pl.loop(0, NB)
        def _(i):
            for j in range(B // L):
                ridx = idx_v[i, pl.ds(j * L, L)]
                nbuf_v[pl.ds(j * L, L)] = plsc.load_gather(norm_f, [ridx])
            pltpu.async_copy(nbuf_v, nr_out.at[w, i], sem_a).wait()

    return sc_normrow


@functools.cache
def _make_sc_message_scatter():
    @functools.partial(
        pl.kernel,
        out_type=jax.ShapeDtypeStruct((NC, NH, NPAD, DQ), jnp.float32),
        mesh=_sc_mesh(),
        compiler_params=pltpu.CompilerParams(needs_layout_passes=False),
        scratch_types=[
            pltpu.VMEM((NB, B), jnp.int32),
            pltpu.VMEM((B,), jnp.int32),
            pltpu.VMEM((B, D), jnp.float32),
            pltpu.VMEM((B // 4, 128), jnp.float32),
            pltpu.VMEM((B, DQ), jnp.float32),
            pltpu.VMEM_SHARED((NPAD, DQ), jnp.float32),
            pltpu.SemaphoreType.DMA,
            pltpu.SemaphoreType.DMA,
            pltpu.SemaphoreType.DMA,
        ],
    )
    def sc_message_scatter(xwb_hbm, ewpk_hbm, rc_hbm, s_out,
                           ridx_v, cidx_v, gbuf, ebuf, mbuf,
                           s_sh, sem_a, sem_b, sem_c):
        c = lax.axis_index("c")
        s = lax.axis_index("s")
        w = s * NC + c
        my_rows = s * ROWS_PER_TILE

        pltpu.sync_copy(rc_hbm.at[w, 0], ridx_v)
        z = jnp.zeros((L,), jnp.float32)

        for h in range(NH):
            @pl.loop(0, B)
            def _(r):
                for f in range(DQ // L):
                    mbuf[r, pl.ds(f * L, L)] = z

            for t in range(ROWS_PER_TILE // B):
                pltpu.sync_copy(mbuf, s_sh.at[pl.ds(my_rows + t * B, B)])
            plsc.subcore_barrier()

            @pl.loop(0, NB)
            def _(i):
                cpc = pltpu.async_copy(rc_hbm.at[w, 1, i], cidx_v, sem_b)
                gather = pltpu.async_copy(
                    xwb_hbm.at[ridx_v.at[i]], gbuf, sem_a)
                base = (h * (EPAD // 4)
                        + w * (EPT_PAD // 4) + i * (B // 4))
                cpe = pltpu.async_copy(
                    ewpk_hbm.at[pl.ds(base, B // 4)], ebuf, sem_c)
                cpc.wait()
                cpe.wait()
                gather.wait()

                for e in range(B):
                    for f in range(DQ // L):
                        zv = (gbuf[e, pl.ds(h * DQ + f * L, L)]
                              + ebuf[e // 4,
                                     pl.ds((e % 4) * DQ + f * L, L)])
                        mbuf[e, pl.ds(f * L, L)] = jnp.maximum(zv, 0.0)

                pltpu.sync_copy(mbuf, s_sh.at[cidx_v], add=True)

            plsc.subcore_barrier()
            pltpu.sync_copy(
                s_sh.at[pl.ds(my_rows, ROWS_PER_TILE)],
                s_out.at[c, h, pl.ds(my_rows, ROWS_PER_TILE)])
            plsc.subcore_barrier()

    return sc_message_scatter


def _tc_xwb_body(x_ref, wx_ref, b_ref, out_ref):
    out_ref[...] = lax.dot_general(
        x_ref[...], wx_ref[...], (((1,), (1,)), ((), ())),
        preferred_element_type=jnp.float32) + b_ref[...]


def _tc_eaw_body(ea_ref, we_ref, out_ref):
    out_ref[...] = lax.dot_general(
        ea_ref[...], we_ref[...], (((1,), (1,)), ((), ())),
        preferred_element_type=jnp.float32)


def _tc_apply_body(x_ref, xwb_ref, la_ref, nrm_ref, s_ref,
                   we_ref, wa_ref, ba_ref, out_ref):
    p = pl.program_id(0)
    selfz = xwb_ref[...] + lax.dot_general(
        la_ref[...], we_ref[...], (((1,), (1,)), ((), ())),
        preferred_element_type=jnp.float32)
    nrm = nrm_ref[...]
    aggr = nrm * s_ref[...] + nrm * nrm * jnp.maximum(selfz, 0.0)
    xa = jnp.concatenate([x_ref[...], aggr], axis=1)
    h = jnp.maximum(
        lax.dot_general(xa, wa_ref[...], (((1,), (1,)), ((), ())),
                        preferred_element_type=jnp.float32) + ba_ref[...],
        0.0)
    rowid = p * 128 + lax.broadcasted_iota(jnp.int32, (128, 1), 0)
    h = jnp.where(rowid < N, h, 0.0)

    @pl.when(p == 0)
    def _():
        out_ref[...] = jnp.zeros_like(out_ref)

    out_ref[...] += jnp.sum(h, axis=0, keepdims=True)


def _tc_head_body(sum_ref, lng_ref, lnb_ref, pw_ref, pb_ref, out_ref):
    gf = sum_ref[...] / jnp.float32(N)
    mu = jnp.mean(gf, axis=-1, keepdims=True)
    var = jnp.mean((gf - mu) ** 2, axis=-1, keepdims=True)
    g2 = (gf - mu) * lax.rsqrt(var + 1e-5) * lng_ref[...] + lnb_ref[...]
    out_ref[...] = jnp.maximum(
        lax.dot_general(g2, pw_ref[...], (((1,), (1,)), ((), ())),
                        preferred_element_type=jnp.float32) + pb_ref[...],
        0.0)


def kernel(x, edge_index, edge_attr, batch, W_msg_w, W_msg_b,
           W_apply_w, W_apply_b, ln_g, ln_b, pred_w, pred_b):
    f32 = jnp.float32
    row0 = edge_index[0]
    col0 = edge_index[1]
    pad_e = EPAD - E

    row_p = jnp.concatenate([row0, jnp.zeros((pad_e,), jnp.int32)])
    col_p = jnp.concatenate([col0, jnp.full((pad_e,), N, jnp.int32)])
    rc1 = jnp.stack([row_p.reshape(NW, EPT_PAD), col_p.reshape(NW, EPT_PAD)],
                    axis=1).reshape(NW, 2, NB, B)
    ea_p = jnp.concatenate([edge_attr, jnp.zeros((pad_e, DE), f32)], axis=0)
    ea_pk = ea_p.reshape(EPAD // 8, 128)
    x_p = jnp.concatenate([x, jnp.zeros((NPAD - N, D), f32)], axis=0)
    Wx = W_msg_w[:, :D]
    We = W_msg_w[:, D:]
    b2 = W_msg_b.reshape(1, D)

    attr_part = _make_sc_degree_attr()(rc1, ea_pk)
    cnt = attr_part[0, :, DE]
    attr_sum = attr_part[0, :, :DE] + attr_part[1, :, :DE]
    deg = cnt + 1.0
    norm = lax.rsqrt(deg)
    loop_attr = attr_sum / jnp.maximum(cnt, 1.0)[:, None]
    nr = _make_sc_normrow()(rc1, norm)

    xwb = pl.pallas_call(
        _tc_xwb_body,
        grid=(NPAD // 256,),
        in_specs=[
            pl.BlockSpec((256, D), lambda p: (p, 0)),
            pl.BlockSpec((D, D), lambda p: (0, 0)),
            pl.BlockSpec((1, D), lambda p: (0, 0)),
        ],
        out_specs=pl.BlockSpec((256, D), lambda p: (p, 0)),
        out_shape=jax.ShapeDtypeStruct((NPAD, D), f32),
    )(x_p, Wx, b2)

    eaw = pl.pallas_call(
        _tc_eaw_body,
        grid=(EPAD // 1024,),
        in_specs=[
            pl.BlockSpec((1024, DE), lambda p: (p, 0)),
            pl.BlockSpec((D, DE), lambda p: (0, 0)),
        ],
        out_specs=pl.BlockSpec((1024, D), lambda p: (p, 0)),
        out_shape=jax.ShapeDtypeStruct((EPAD, D), f32),
    )(ea_p, We)
    xwn = xwb * norm[:, None]
    eawn = eaw * nr.reshape(EPAD)[:, None]
    ew_pk = jnp.transpose(
        eawn.reshape(EPAD, NH, DQ), (1, 0, 2)).reshape(NH * (EPAD // 4), 128)

    s_part = _make_sc_message_scatter()(xwn, ew_pk, rc1)
    s_sum = s_part[0] + s_part[1]
    s_full = jnp.transpose(s_sum, (1, 0, 2)).reshape(NPAD, D)

    hsum = pl.pallas_call(
        _tc_apply_body,
        grid=(NPAD // 128,),
        in_specs=[
            pl.BlockSpec((128, D), lambda p: (p, 0)),
            pl.BlockSpec((128, D), lambda p: (p, 0)),
            pl.BlockSpec((128, DE), lambda p: (p, 0)),
            pl.BlockSpec((128, 1), lambda p: (p, 0)),
            pl.BlockSpec((128, D), lambda p: (p, 0)),
            pl.BlockSpec((D, DE), lambda p: (0, 0)),
            pl.BlockSpec((H1, 2 * D), lambda p: (0, 0)),
            pl.BlockSpec((1, H1), lambda p: (0, 0)),
        ],
        out_specs=pl.BlockSpec((1, H1), lambda p: (0, 0)),
        out_shape=jax.ShapeDtypeStruct((1, H1), f32),
    )(x_p, xwb, loop_attr, norm[:, None], s_full,
      We, W_apply_w, W_apply_b.reshape(1, H1))

    out = pl.pallas_call(
        _tc_head_body,
        in_specs=[
            pl.BlockSpec((1, H1), lambda: (0, 0)),
            pl.BlockSpec((1, H1), lambda: (0, 0)),
            pl.BlockSpec((1, H1), lambda: (0, 0)),
            pl.BlockSpec((PH, H1), lambda: (0, 0)),
            pl.BlockSpec((1, PH), lambda: (0, 0)),
        ],
        out_specs=pl.BlockSpec((1, PH), lambda: (0, 0)),
        out_shape=jax.ShapeDtypeStruct((1, PH), f32),
    )(hsum, ln_g.reshape(1, H1), ln_b.reshape(1, H1),
      pred_w, pred_b.reshape(1, PH))

    return out
